# Initial kernel scaffold; baseline (speedup 1.0000x reference)
import jax
import jax.numpy as jnp
from jax.experimental import pallas as pl


def kernel(x, W_router, attn_norm_w, ssm_norm_w, Wq, Wk, Wv, Wo, W_gate, W_up, W_down):
    raise NotImplementedError("write your pallas kernel here")


# speedup vs baseline: 1.4406x; 1.4406x over previous
import functools

import jax
import jax.numpy as jnp
from jax.experimental import pallas as pl
from jax.experimental.pallas import tpu as pltpu

NHEAD = 16
CAP_FACTOR = 1.25
TEMP = 1.0
EPS = 1e-6

BM = 512
BQ = 512
BK = 512


def _bf16_dot(a, b):
    return jax.lax.dot_general(
        a.astype(jnp.bfloat16), b.astype(jnp.bfloat16),
        (((1,), (0,)), ((), ())), preferred_element_type=jnp.float32)


def _rmsnorm_rows(x_blk, norm_w):
    var = jnp.mean(x_blk * x_blk, axis=-1, keepdims=True)
    return x_blk * jax.lax.rsqrt(var + EPS) * norm_w


def _qkv_body(x_ref, nw_ref, w_ref, out_ref):
    xn = _rmsnorm_rows(x_ref[...], nw_ref[0, :])
    out_ref[...] = _bf16_dot(xn, w_ref[0]).astype(jnp.bfloat16)


def _qkv_proj(x2, attn_norm_w, w_stack, M, H):
    n_w = w_stack.shape[0]
    return pl.pallas_call(
        _qkv_body,
        grid=(n_w, M // BM),
        in_specs=[
            pl.BlockSpec((BM, H), lambda n, m: (m, 0)),
            pl.BlockSpec((1, H), lambda n, m: (0, 0)),
            pl.BlockSpec((1, H, H), lambda n, m: (n, 0, 0)),
        ],
        out_specs=pl.BlockSpec((BM, H), lambda n, m: (m, n)),
        out_shape=jax.ShapeDtypeStruct((M, n_w * H), jnp.bfloat16),
    )(x2, attn_norm_w.reshape(1, H), w_stack)


def _flash_body(q_ref, k_ref, v_ref, o_ref, *, bq, bk, d, scale):
    qi = pl.program_id(1)
    q = q_ref[...]

    def step(kb, carry):
        acc, m_i, l_i = carry
        k_blk = k_ref[pl.ds(kb * bk, bk), :]
        v_blk = v_ref[pl.ds(kb * bk, bk), :]
        s = jax.lax.dot_general(
            q, k_blk, (((1,), (1,)), ((), ())),
            preferred_element_type=jnp.float32) * scale
        row = qi * bq + jax.lax.broadcasted_iota(jnp.int32, (bq, bk), 0)
        col = kb * bk + jax.lax.broadcasted_iota(jnp.int32, (bq, bk), 1)
        s = jnp.where(row >= col, s, -1e9)
        m_new = jnp.maximum(m_i, jnp.max(s, axis=-1, keepdims=True))
        alpha = jnp.exp(m_i - m_new)
        p = jnp.exp(s - m_new)
        l_new = l_i * alpha + jnp.sum(p, axis=-1, keepdims=True)
        acc_new = acc * alpha + jax.lax.dot_general(
            p.astype(jnp.bfloat16), v_blk, (((1,), (0,)), ((), ())),
            preferred_element_type=jnp.float32)
        return acc_new, m_new, l_new

    acc0 = jnp.zeros((bq, d), jnp.float32)
    m0 = jnp.full((bq, 1), -1e30, jnp.float32)
    l0 = jnp.zeros((bq, 1), jnp.float32)
    acc, m_i, l_i = jax.lax.fori_loop(0, qi + 1, step, (acc0, m0, l0))
    o_ref[...] = (acc / l_i).astype(jnp.bfloat16)


def _flash_attn(qkv, B, S, H):
    d = H // NHEAD
    nq = S // BQ
    body = functools.partial(_flash_body, bq=BQ, bk=BK, d=d,
                             scale=1.0 / (d ** 0.5))
    ctx = pl.pallas_call(
        body,
        grid=(B * NHEAD, nq),
        in_specs=[
            pl.BlockSpec((BQ, d),
                         lambda bh, qi: ((bh // NHEAD) * (S // BQ) + qi,
                                         bh % NHEAD)),
            pl.BlockSpec((S, d),
                         lambda bh, qi: (bh // NHEAD, NHEAD + bh % NHEAD)),
            pl.BlockSpec((S, d),
                         lambda bh, qi: (bh // NHEAD, 2 * NHEAD + bh % NHEAD)),
        ],
        out_specs=pl.BlockSpec((BQ, d),
                               lambda bh, qi: ((bh // NHEAD) * (S // BQ) + qi,
                                               bh % NHEAD)),
        out_shape=jax.ShapeDtypeStruct((B * S, H), jnp.bfloat16),
    )(qkv, qkv, qkv)
    return ctx


def _swiglu_body(x_ref, nw_ref, wg_ref, wu_ref, out_ref):
    xn = _rmsnorm_rows(x_ref[...], nw_ref[0, :])
    g = _bf16_dot(xn, wg_ref[...])
    u = _bf16_dot(xn, wu_ref[...])
    out_ref[...] = (g * jax.nn.sigmoid(g) * u).astype(jnp.bfloat16)


def _swiglu_mid(x2, ssm_norm_w, wg_bf, wu_bf, M, H):
    return pl.pallas_call(
        _swiglu_body,
        grid=(M // BM,),
        in_specs=[
            pl.BlockSpec((BM, H), lambda m: (m, 0)),
            pl.BlockSpec((1, H), lambda m: (0, 0)),
            pl.BlockSpec((H, H), lambda m: (0, 0)),
            pl.BlockSpec((H, H), lambda m: (0, 0)),
        ],
        out_specs=pl.BlockSpec((BM, H), lambda m: (m, 0)),
        out_shape=jax.ShapeDtypeStruct((M, H), jnp.bfloat16),
    )(x2, ssm_norm_w.reshape(1, H), wg_bf, wu_bf)


def _comb_body(ctx_ref, h_ref, wo_ref, wd_ref, out_ref):
    out_ref[...] = (_bf16_dot(ctx_ref[...], wo_ref[...])
                    + _bf16_dot(h_ref[...], wd_ref[...]))


def _combine(ctx, h, wo_bf, wd_bf, M, H):
    return pl.pallas_call(
        _comb_body,
        grid=(M // BM,),
        in_specs=[
            pl.BlockSpec((BM, H), lambda m: (m, 0)),
            pl.BlockSpec((BM, H), lambda m: (m, 0)),
            pl.BlockSpec((H, H), lambda m: (0, 0)),
            pl.BlockSpec((H, H), lambda m: (0, 0)),
        ],
        out_specs=pl.BlockSpec((BM, H), lambda m: (m, 0)),
        out_shape=jax.ShapeDtypeStruct((M, H), jnp.float32),
    )(ctx, h, wo_bf, wd_bf)


def kernel(x, W_router, attn_norm_w, ssm_norm_w, Wq, Wk, Wv, Wo,
           W_gate, W_up, W_down):
    B, S, H = x.shape
    M = B * S
    x2 = x.reshape(M, H)

    w_stack = jnp.stack([Wq, Wk, Wv]).astype(jnp.bfloat16)
    wg_bf = W_gate.astype(jnp.bfloat16)
    wu_bf = W_up.astype(jnp.bfloat16)
    wo_bf = Wo.astype(jnp.bfloat16)
    wd_bf = W_down.astype(jnp.bfloat16)

    qkv = _qkv_proj(x2, attn_norm_w, w_stack, M, H)
    ctx = _flash_attn(qkv, B, S, H)
    h = _swiglu_mid(x2, ssm_norm_w, wg_bf, wu_bf, M, H)
    out = _combine(ctx, h, wo_bf, wd_bf, M, H).reshape(B, S, H)

    router_logits = (x @ W_router) / TEMP
    router_probs = jax.nn.softmax(router_logits, axis=-1)
    attention_prob = router_probs[..., 0]
    ssm_prob = router_probs[..., 1]
    balance_loss = (jnp.var(attention_prob.mean(axis=1), ddof=1)
                    + jnp.var(ssm_prob.mean(axis=1), ddof=1))
    routing_loss = balance_loss * 0.1
    return out, routing_loss


# speedup vs baseline: 1.4527x; 1.0084x over previous
import functools

import jax
import jax.numpy as jnp
from jax.experimental import pallas as pl
from jax.experimental.pallas import tpu as pltpu

NHEAD = 16
CAP_FACTOR = 1.25
TEMP = 1.0
EPS = 1e-6

BM = 512
BQ = 512
BK = 512


def _bf16_dot(a, b):
    return jax.lax.dot_general(
        a.astype(jnp.bfloat16), b.astype(jnp.bfloat16),
        (((1,), (0,)), ((), ())), preferred_element_type=jnp.float32)


def _rmsnorm_rows(x_blk, norm_w):
    var = jnp.mean(x_blk * x_blk, axis=-1, keepdims=True)
    return x_blk * jax.lax.rsqrt(var + EPS) * norm_w


def _qkv_body(x_ref, nw_ref, wq_ref, wk_ref, wv_ref, out_ref):
    xn = _rmsnorm_rows(x_ref[...], nw_ref[0, :]).astype(jnp.bfloat16)
    h = wq_ref.shape[0]
    out_ref[:, 0:h] = _bf16_dot(xn, wq_ref[...]).astype(jnp.bfloat16)
    out_ref[:, h:2 * h] = _bf16_dot(xn, wk_ref[...]).astype(jnp.bfloat16)
    out_ref[:, 2 * h:3 * h] = _bf16_dot(xn, wv_ref[...]).astype(jnp.bfloat16)


def _qkv_proj(x2, attn_norm_w, wq_bf, wk_bf, wv_bf, M, H):
    return pl.pallas_call(
        _qkv_body,
        grid=(M // BM,),
        in_specs=[
            pl.BlockSpec((BM, H), lambda m: (m, 0)),
            pl.BlockSpec((1, H), lambda m: (0, 0)),
            pl.BlockSpec((H, H), lambda m: (0, 0)),
            pl.BlockSpec((H, H), lambda m: (0, 0)),
            pl.BlockSpec((H, H), lambda m: (0, 0)),
        ],
        out_specs=pl.BlockSpec((BM, 3 * H), lambda m: (m, 0)),
        out_shape=jax.ShapeDtypeStruct((M, 3 * H), jnp.bfloat16),
    )(x2, attn_norm_w.reshape(1, H), wq_bf, wk_bf, wv_bf)


def _flash_body(q_ref, k_ref, v_ref, o_ref, *, bq, bk, d, scale):
    qi = pl.program_id(1)
    q = q_ref[...]

    def step(kb, carry):
        acc, m_i, l_i = carry
        k_blk = k_ref[pl.ds(kb * bk, bk), :]
        v_blk = v_ref[pl.ds(kb * bk, bk), :]
        s = jax.lax.dot_general(
            q, k_blk, (((1,), (1,)), ((), ())),
            preferred_element_type=jnp.float32) * scale
        row = qi * bq + jax.lax.broadcasted_iota(jnp.int32, (bq, bk), 0)
        col = kb * bk + jax.lax.broadcasted_iota(jnp.int32, (bq, bk), 1)
        s = jnp.where(row >= col, s, -1e9)
        m_new = jnp.maximum(m_i, jnp.max(s, axis=-1, keepdims=True))
        alpha = jnp.exp(m_i - m_new)
        p = jnp.exp(s - m_new)
        l_new = l_i * alpha + jnp.sum(p, axis=-1, keepdims=True)
        acc_new = acc * alpha + jax.lax.dot_general(
            p.astype(jnp.bfloat16), v_blk, (((1,), (0,)), ((), ())),
            preferred_element_type=jnp.float32)
        return acc_new, m_new, l_new

    acc0 = jnp.zeros((bq, d), jnp.float32)
    m0 = jnp.full((bq, 1), -1e30, jnp.float32)
    l0 = jnp.zeros((bq, 1), jnp.float32)
    acc, m_i, l_i = jax.lax.fori_loop(0, qi + 1, step, (acc0, m0, l0))
    o_ref[...] = (acc / l_i).astype(jnp.bfloat16)


def _flash_attn(qkv, B, S, H):
    d = H // NHEAD
    nq = S // BQ
    body = functools.partial(_flash_body, bq=BQ, bk=BK, d=d,
                             scale=1.0 / (d ** 0.5))
    ctx = pl.pallas_call(
        body,
        grid=(B * NHEAD, nq),
        in_specs=[
            pl.BlockSpec((BQ, d),
                         lambda bh, qi: ((bh // NHEAD) * (S // BQ) + qi,
                                         bh % NHEAD)),
            pl.BlockSpec((S, d),
                         lambda bh, qi: (bh // NHEAD, NHEAD + bh % NHEAD)),
            pl.BlockSpec((S, d),
                         lambda bh, qi: (bh // NHEAD, 2 * NHEAD + bh % NHEAD)),
        ],
        out_specs=pl.BlockSpec((BQ, d),
                               lambda bh, qi: ((bh // NHEAD) * (S // BQ) + qi,
                                               bh % NHEAD)),
        out_shape=jax.ShapeDtypeStruct((B * S, H), jnp.bfloat16),
    )(qkv, qkv, qkv)
    return ctx


def _swiglu_body(x_ref, nw_ref, wg_ref, wu_ref, out_ref):
    xn = _rmsnorm_rows(x_ref[...], nw_ref[0, :])
    g = _bf16_dot(xn, wg_ref[...])
    u = _bf16_dot(xn, wu_ref[...])
    out_ref[...] = (g * jax.nn.sigmoid(g) * u).astype(jnp.bfloat16)


def _swiglu_mid(x2, ssm_norm_w, wg_bf, wu_bf, M, H):
    return pl.pallas_call(
        _swiglu_body,
        grid=(M // BM,),
        in_specs=[
            pl.BlockSpec((BM, H), lambda m: (m, 0)),
            pl.BlockSpec((1, H), lambda m: (0, 0)),
            pl.BlockSpec((H, H), lambda m: (0, 0)),
            pl.BlockSpec((H, H), lambda m: (0, 0)),
        ],
        out_specs=pl.BlockSpec((BM, H), lambda m: (m, 0)),
        out_shape=jax.ShapeDtypeStruct((M, H), jnp.bfloat16),
    )(x2, ssm_norm_w.reshape(1, H), wg_bf, wu_bf)


def _comb_body(ctx_ref, h_ref, wo_ref, wd_ref, out_ref):
    out_ref[...] = (_bf16_dot(ctx_ref[...], wo_ref[...])
                    + _bf16_dot(h_ref[...], wd_ref[...]))


def _combine(ctx, h, wo_bf, wd_bf, M, H):
    return pl.pallas_call(
        _comb_body,
        grid=(M // BM,),
        in_specs=[
            pl.BlockSpec((BM, H), lambda m: (m, 0)),
            pl.BlockSpec((BM, H), lambda m: (m, 0)),
            pl.BlockSpec((H, H), lambda m: (0, 0)),
            pl.BlockSpec((H, H), lambda m: (0, 0)),
        ],
        out_specs=pl.BlockSpec((BM, H), lambda m: (m, 0)),
        out_shape=jax.ShapeDtypeStruct((M, H), jnp.float32),
    )(ctx, h, wo_bf, wd_bf)


def kernel(x, W_router, attn_norm_w, ssm_norm_w, Wq, Wk, Wv, Wo,
           W_gate, W_up, W_down):
    B, S, H = x.shape
    M = B * S
    x2 = x.reshape(M, H)

    wq_bf = Wq.astype(jnp.bfloat16)
    wk_bf = Wk.astype(jnp.bfloat16)
    wv_bf = Wv.astype(jnp.bfloat16)
    wg_bf = W_gate.astype(jnp.bfloat16)
    wu_bf = W_up.astype(jnp.bfloat16)
    wo_bf = Wo.astype(jnp.bfloat16)
    wd_bf = W_down.astype(jnp.bfloat16)

    qkv = _qkv_proj(x2, attn_norm_w, wq_bf, wk_bf, wv_bf, M, H)
    ctx = _flash_attn(qkv, B, S, H)
    h = _swiglu_mid(x2, ssm_norm_w, wg_bf, wu_bf, M, H)
    out = _combine(ctx, h, wo_bf, wd_bf, M, H).reshape(B, S, H)

    router_logits = (x @ W_router) / TEMP
    router_probs = jax.nn.softmax(router_logits, axis=-1)
    attention_prob = router_probs[..., 0]
    ssm_prob = router_probs[..., 1]
    balance_loss = (jnp.var(attention_prob.mean(axis=1), ddof=1)
                    + jnp.var(ssm_prob.mean(axis=1), ddof=1))
    routing_loss = balance_loss * 0.1
    return out, routing_loss
